# baseline (speedup 1.0000x reference)
import functools

import jax
import jax.numpy as jnp
from jax.experimental import pallas as pl
from jax.experimental.pallas import tpu as pltpu

N_NODES = 10000
N_EDGES = 320000
NODE_DIM = 128
EDGE_DIM = 16
HIDDEN = 128

E_BLK = 10000


def _node_mlp_kernel(x_ref, w1a_ref, w1b_ref, b1_ref, xa_ref, xb_ref):
    x = x_ref[...]
    xa_ref[...] = jnp.dot(x, w1a_ref[...], preferred_element_type=jnp.float32) + b1_ref[...]
    xb_ref[...] = jnp.dot(x, w1b_ref[...], preferred_element_type=jnp.float32)


def _edge_mlp_kernel(ea_ref, w1c_ref, out_ref):
    out_ref[...] = jnp.dot(ea_ref[...], w1c_ref[...], preferred_element_type=jnp.float32)


def _edge_loop_kernel(dst_ref, src_ref, ea_ref, xa_ref, xb_ref, hacc_ref, cnt_ref):
    k = pl.program_id(0)

    @pl.when(k == 0)
    def _():
        hacc_ref[...] = jnp.zeros_like(hacc_ref)
        cnt_ref[...] = jnp.zeros_like(cnt_ref)

    def body(i, carry):
        d = dst_ref[0, 0, i]
        s = src_ref[0, 0, i]
        h = jnp.maximum(
            xa_ref[pl.ds(d, 1), :] + xb_ref[pl.ds(s, 1), :] + ea_ref[pl.ds(i, 1), :],
            0.0,
        )
        hacc_ref[pl.ds(d, 1), :] += h
        cnt_ref[pl.ds(d, 1), :] += 1.0
        return carry

    jax.lax.fori_loop(0, E_BLK, body, 0)


def _final_kernel(hacc_ref, cnt_ref, w2_ref, b2_ref, out_ref):
    out_ref[...] = (
        jnp.dot(hacc_ref[...], w2_ref[...], preferred_element_type=jnp.float32)
        + cnt_ref[:, 0:1] * b2_ref[...]
    )


def kernel(x, edge_index, edge_attr, W1, b1, W2, b2):
    dst = edge_index[1].astype(jnp.int32)
    src = edge_index[0].astype(jnp.int32)
    W1a = W1[:NODE_DIM]
    W1b = W1[NODE_DIM:2 * NODE_DIM]
    W1c = W1[2 * NODE_DIM:]
    b1r = b1.reshape(1, HIDDEN)
    b2r = b2.reshape(1, NODE_DIM)

    xa, xb = pl.pallas_call(
        _node_mlp_kernel,
        out_shape=[
            jax.ShapeDtypeStruct((N_NODES, HIDDEN), jnp.float32),
            jax.ShapeDtypeStruct((N_NODES, HIDDEN), jnp.float32),
        ],
    )(x, W1a, W1b, b1r)

    ea = pl.pallas_call(
        _edge_mlp_kernel,
        grid=(N_EDGES // 20000,),
        in_specs=[
            pl.BlockSpec((20000, EDGE_DIM), lambda k: (k, 0)),
            pl.BlockSpec((EDGE_DIM, HIDDEN), lambda k: (0, 0)),
        ],
        out_specs=pl.BlockSpec((20000, HIDDEN), lambda k: (k, 0)),
        out_shape=jax.ShapeDtypeStruct((N_EDGES, HIDDEN), jnp.float32),
    )(edge_attr, W1c)

    nblk = N_EDGES // E_BLK
    dst3 = dst.reshape(nblk, 1, E_BLK)
    src3 = src.reshape(nblk, 1, E_BLK)
    hacc, cnt = pl.pallas_call(
        _edge_loop_kernel,
        grid=(nblk,),
        in_specs=[
            pl.BlockSpec((1, 1, E_BLK), lambda k: (k, 0, 0), memory_space=pltpu.SMEM),
            pl.BlockSpec((1, 1, E_BLK), lambda k: (k, 0, 0), memory_space=pltpu.SMEM),
            pl.BlockSpec((E_BLK, HIDDEN), lambda k: (k, 0)),
            pl.BlockSpec((N_NODES, HIDDEN), lambda k: (0, 0)),
            pl.BlockSpec((N_NODES, HIDDEN), lambda k: (0, 0)),
        ],
        out_specs=[
            pl.BlockSpec((N_NODES, HIDDEN), lambda k: (0, 0)),
            pl.BlockSpec((N_NODES, 128), lambda k: (0, 0)),
        ],
        out_shape=[
            jax.ShapeDtypeStruct((N_NODES, HIDDEN), jnp.float32),
            jax.ShapeDtypeStruct((N_NODES, 128), jnp.float32),
        ],
    )(dst3, src3, ea, xa, xb)

    out = pl.pallas_call(
        _final_kernel,
        out_shape=jax.ShapeDtypeStruct((N_NODES, NODE_DIM), jnp.float32),
    )(hacc, cnt, W2, b2r)
    return out


# speedup vs baseline: 3.9547x; 3.9547x over previous
import functools

import jax
import jax.numpy as jnp
from jax import lax
from jax.experimental import pallas as pl
from jax.experimental.pallas import tpu as pltpu
from jax.experimental.pallas import tpu_sc as plsc

N_NODES = 10000
N_EDGES = 320000
NODE_DIM = 128
EDGE_DIM = 16
HIDDEN = 128

NCORES = 2
NSUB = 16
NTILES = NCORES * NSUB
EDGES_PER_TILE = N_EDGES // NTILES
CB = 80
NCHUNKS = EDGES_PER_TILE // CB
ROWS_PER_TILE = N_NODES // NSUB
CW = 16


def _node_mlp_kernel(x_ref, w1a_ref, w1b_ref, b1_ref, xa_ref, xb_ref):
    x = x_ref[...]
    xa_ref[...] = jnp.dot(x, w1a_ref[...], preferred_element_type=jnp.float32) + b1_ref[...]
    xb_ref[...] = jnp.dot(x, w1b_ref[...], preferred_element_type=jnp.float32)


def _edge_mlp_kernel(ea_ref, w1c_ref, out_ref):
    out_ref[...] = jnp.dot(ea_ref[...], w1c_ref[...], preferred_element_type=jnp.float32)


def _sc_count_kernel(dst_hbm, cnt_hbm, idxd, ones_b, cnt_sh, sem):
    c = lax.axis_index("c")
    s = lax.axis_index("s")
    tile = c * NSUB + s
    ebase = tile * EDGES_PER_TILE
    rbase = s * 640

    @pl.loop(0, CB)
    def _(r):
        ones_b.at[pl.ds(r, 1), pl.ds(0, CW)][...] = jnp.zeros((1, CW), jnp.float32)
    for k in range(8):
        @pl.when(rbase + k * CB < N_NODES)
        def _(k=k):
            pltpu.sync_copy(ones_b, cnt_sh.at[pl.ds(rbase + k * CB, CB), :])

    @pl.loop(0, CB)
    def _(r):
        ones_b.at[pl.ds(r, 1), pl.ds(0, CW)][...] = jnp.full((1, CW), 1.0, jnp.float32)
    plsc.subcore_barrier()

    @pl.loop(0, NCHUNKS)
    def _(j):
        pltpu.sync_copy(dst_hbm.at[pl.ds(ebase + j * CB, CB)], idxd)
        pltpu.sync_copy(ones_b, cnt_sh.at[idxd], add=True)

    plsc.subcore_barrier()
    for k in range(8):
        @pl.when(rbase + k * CB < N_NODES)
        def _(k=k):
            pltpu.sync_copy(cnt_sh.at[pl.ds(rbase + k * CB, CB), :],
                            cnt_hbm.at[pl.ds(c * N_NODES + rbase + k * CB, CB), :])


def _sc_edge_kernel(dst_hbm, src_hbm, ea_hbm, xa_hbm, xb_hbm,
                    hacc_hbm,
                    idxd, idxs, bufa, bufb, bufe,
                    acc_sh, sem_a, sem_b, sem_e):
    c = lax.axis_index("c")
    s = lax.axis_index("s")
    tile = c * NSUB + s
    ebase = tile * EDGES_PER_TILE

    @pl.loop(0, CB)
    def _(r):
        for cc in range(NODE_DIM // 16):
            bufa.at[pl.ds(r, 1), pl.ds(cc * 16, 16)][...] = jnp.zeros((1, 16), jnp.float32)

    rbase = s * 640
    for k in range(8):
        @pl.when(rbase + k * CB < N_NODES)
        def _(k=k):
            pltpu.sync_copy(bufa, acc_sh.at[pl.ds(rbase + k * CB, CB), :])
    plsc.subcore_barrier()

    @pl.loop(0, NCHUNKS)
    def _(j):
        base = ebase + j * CB
        pltpu.sync_copy(dst_hbm.at[pl.ds(base, CB)], idxd)
        pltpu.sync_copy(src_hbm.at[pl.ds(base, CB)], idxs)
        ca = pltpu.async_copy(xa_hbm.at[idxd], bufa, sem_a)
        cb = pltpu.async_copy(xb_hbm.at[idxs], bufb, sem_b)
        ce = pltpu.async_copy(ea_hbm.at[pl.ds(base, CB), :], bufe, sem_e)
        ca.wait()
        cb.wait()
        ce.wait()

        @pl.loop(0, CB)
        def _(r):
            for cc in range(NODE_DIM // 16):
                slc = (pl.ds(r, 1), pl.ds(cc * 16, 16))
                bufa.at[slc][...] = jnp.maximum(
                    bufa.at[slc][...] + bufb.at[slc][...] + bufe.at[slc][...], 0.0
                )

        pltpu.sync_copy(bufa, acc_sh.at[idxd], add=True)

    plsc.subcore_barrier()
    for k in range(8):
        @pl.when(rbase + k * CB < N_NODES)
        def _(k=k):
            pltpu.sync_copy(acc_sh.at[pl.ds(rbase + k * CB, CB), :],
                            hacc_hbm.at[pl.ds(c * N_NODES + rbase + k * CB, CB), :])


def _combine_kernel(hacc_ref, cnt_ref, w2_ref, b2_ref, out_ref):
    h = hacc_ref[:N_NODES, :] + hacc_ref[N_NODES:, :]
    cnt = cnt_ref[:N_NODES, 0:1] + cnt_ref[N_NODES:, 0:1]
    out_ref[...] = (
        jnp.dot(h, w2_ref[...], preferred_element_type=jnp.float32) + cnt * b2_ref[...]
    )


def kernel(x, edge_index, edge_attr, W1, b1, W2, b2):
    dst = edge_index[1].astype(jnp.int32)
    src = edge_index[0].astype(jnp.int32)
    W1a = W1[:NODE_DIM]
    W1b = W1[NODE_DIM:2 * NODE_DIM]
    W1c = W1[2 * NODE_DIM:]
    b1r = b1.reshape(1, HIDDEN)
    b2r = b2.reshape(1, NODE_DIM)

    xa, xb = pl.pallas_call(
        _node_mlp_kernel,
        out_shape=[
            jax.ShapeDtypeStruct((N_NODES, HIDDEN), jnp.float32),
            jax.ShapeDtypeStruct((N_NODES, HIDDEN), jnp.float32),
        ],
    )(x, W1a, W1b, b1r)

    ea = pl.pallas_call(
        _edge_mlp_kernel,
        grid=(N_EDGES // 20000,),
        in_specs=[
            pl.BlockSpec((20000, EDGE_DIM), lambda k: (k, 0)),
            pl.BlockSpec((EDGE_DIM, HIDDEN), lambda k: (0, 0)),
        ],
        out_specs=pl.BlockSpec((20000, HIDDEN), lambda k: (k, 0)),
        out_shape=jax.ShapeDtypeStruct((N_EDGES, HIDDEN), jnp.float32),
    )(edge_attr, W1c)

    mesh = plsc.VectorSubcoreMesh(core_axis_name="c", subcore_axis_name="s")
    sc_count = pl.kernel(
        _sc_count_kernel,
        out_type=jax.ShapeDtypeStruct((NCORES * N_NODES, CW), jnp.float32),
        mesh=mesh,
        scratch_types=[
            pltpu.VMEM((CB,), jnp.int32),
            pltpu.VMEM((CB, CW), jnp.float32),
            pltpu.VMEM_SHARED((N_NODES, CW), jnp.float32),
            pltpu.SemaphoreType.DMA,
        ],
    )
    cnt = sc_count(dst)

    sc_edge = pl.kernel(
        _sc_edge_kernel,
        out_type=jax.ShapeDtypeStruct((NCORES * N_NODES, HIDDEN), jnp.float32),
        mesh=mesh,
        scratch_types=[
            pltpu.VMEM((CB,), jnp.int32),
            pltpu.VMEM((CB,), jnp.int32),
            pltpu.VMEM((CB, HIDDEN), jnp.float32),
            pltpu.VMEM((CB, HIDDEN), jnp.float32),
            pltpu.VMEM((CB, HIDDEN), jnp.float32),
            pltpu.VMEM_SHARED((N_NODES, HIDDEN), jnp.float32),
            pltpu.SemaphoreType.DMA,
            pltpu.SemaphoreType.DMA,
            pltpu.SemaphoreType.DMA,
        ],
    )
    hacc = sc_edge(dst, src, ea, xa, xb)

    out = pl.pallas_call(
        _combine_kernel,
        out_shape=jax.ShapeDtypeStruct((N_NODES, NODE_DIM), jnp.float32),
    )(hacc, cnt, W2, b2r)
    return out
